# Initial kernel scaffold; baseline (speedup 1.0000x reference)
import jax
import jax.numpy as jnp
from jax import lax
from jax.experimental import pallas as pl
from jax.experimental.pallas import tpu as pltpu
from jax.experimental.pallas import tpu_sc as plsc

_NUM_CLASSES = 1000
_DIM = 128
_N = 320000
_MOMENTUM = 0.99

_NC = 2
_NS = 16
_NW = _NC * _NS
_G = 128
_NG = _N // _G
_CPAD = 1024
_CPS = _CPAD // _NS


def _phase1_body(emb_hbm, lab_hbm, sums_out, counts_out,
                 rows_v, lab_v, ones_v, zsum_v, zcnt_v, sums_sp, counts_sp):
    c = lax.axis_index("c")
    s = lax.axis_index("s")
    wid = s * _NC + c

    def _fill_ones(i, _):
        ones_v[i] = jnp.ones((16,), jnp.float32)
        return 0
    lax.fori_loop(0, _G, _fill_ones, 0)

    def _fill_zsum(i, _):
        zsum_v[i // 8, pl.ds((i % 8) * 16, 16)] = jnp.zeros((16,), jnp.float32)
        return 0
    lax.fori_loop(0, _CPS * 8, _fill_zsum, 0)

    def _fill_zcnt(i, _):
        zcnt_v[i] = jnp.zeros((16,), jnp.float32)
        return 0
    lax.fori_loop(0, _CPS, _fill_zcnt, 0)

    base = s * _CPS
    pltpu.sync_copy(zsum_v, sums_sp.at[pl.ds(base, _CPS)])
    pltpu.sync_copy(zcnt_v, counts_sp.at[pl.ds(base, _CPS)])
    plsc.subcore_barrier()

    g0 = wid * _NG // _NW
    g1 = (wid + 1) * _NG // _NW

    def _chunk(g, _):
        pltpu.sync_copy(emb_hbm.at[pl.ds(g * _G, _G)], rows_v)
        pltpu.sync_copy(lab_hbm.at[g], lab_v.at[0])
        pltpu.sync_copy(rows_v, sums_sp.at[lab_v.at[0]], add=True)
        pltpu.sync_copy(ones_v, counts_sp.at[lab_v.at[0]], add=True)
        return 0
    lax.fori_loop(g0, g1, _chunk, 0)

    plsc.subcore_barrier()

    pltpu.sync_copy(sums_sp.at[pl.ds(base, _CPS)], zsum_v)
    pltpu.sync_copy(zsum_v, sums_out.at[c, pl.ds(base, _CPS)])
    pltpu.sync_copy(counts_sp.at[pl.ds(base, _CPS)], zcnt_v)
    pltpu.sync_copy(zcnt_v, counts_out.at[c, pl.ds(base, _CPS)])


_phase1 = pl.kernel(
    _phase1_body,
    out_type=(
        jax.ShapeDtypeStruct((_NC, _CPAD, _DIM), jnp.float32),
        jax.ShapeDtypeStruct((_NC, _CPAD, 16), jnp.float32),
    ),
    mesh=plsc.VectorSubcoreMesh(
        core_axis_name="c", subcore_axis_name="s",
        num_cores=_NC, num_subcores=_NS),
    scratch_types=(
        pltpu.VMEM((_G, _DIM), jnp.float32),
        pltpu.VMEM((2, _G), jnp.int32),
        pltpu.VMEM((_G, 16), jnp.float32),
        pltpu.VMEM((_CPS, _DIM), jnp.float32),
        pltpu.VMEM((_CPS, 16), jnp.float32),
        pltpu.VMEM_SHARED((_CPAD, _DIM), jnp.float32),
        pltpu.VMEM_SHARED((_CPAD, 16), jnp.float32),
    ),
)


def _combine_body(sums_ref, counts_ref, proto_ref, out_ref):
    total = sums_ref[0] + sums_ref[1]
    cnt = counts_ref[0, :, 0:1] + counts_ref[1, :, 0:1]
    mean = total / jnp.maximum(cnt, 1.0)
    proto = proto_ref[...]
    out_ref[...] = jnp.where(
        cnt > 0.0, _MOMENTUM * proto + (1.0 - _MOMENTUM) * mean, proto)


def kernel(embeddings, labels, prototypes):
    lab2d = labels.astype(jnp.int32).reshape(_NG, _G)
    sums, counts = _phase1(embeddings, lab2d)
    return pl.pallas_call(
        _combine_body,
        out_shape=jax.ShapeDtypeStruct((_NUM_CLASSES, _DIM), jnp.float32),
    )(sums[:, :_NUM_CLASSES], counts[:, :_NUM_CLASSES], prototypes)


# speedup vs baseline: 7.4573x; 7.4573x over previous
import jax
import jax.numpy as jnp
from jax import lax
from jax.experimental import pallas as pl
from jax.experimental.pallas import tpu as pltpu
from jax.experimental.pallas import tpu_sc as plsc

_NUM_CLASSES = 1000
_DIM = 128
_N = 320000
_MOMENTUM = 0.99

_NC = 2
_NS = 16
_NW = _NC * _NS
_G = 128
_NG = _N // _G
_CPAD = 1024
_CPS = _CPAD // _NS
_W = _DIM


def _phase1_body(emb_hbm, lab_hbm, ones_hbm, zero_hbm, sums_out, counts_out,
                 rows_v, lab_v, ones_v, stage_v, sums_sp, counts_sp,
                 sem0, sem1):
    c = lax.axis_index("c")
    s = lax.axis_index("s")
    wid = s * _NC + c

    pltpu.sync_copy(ones_hbm, ones_v)

    base = s * _CPS
    pltpu.sync_copy(zero_hbm, stage_v)
    pltpu.sync_copy(stage_v, sums_sp.at[pl.ds(base, _CPS)])
    pltpu.sync_copy(stage_v, counts_sp.at[pl.ds(base, _CPS)])
    plsc.subcore_barrier()

    g0 = wid * _NG // _NW
    g1 = (wid + 1) * _NG // _NW
    sems = (sem0, sem1)

    def _start_load(g, b, sem):
        pltpu.async_copy(emb_hbm.at[pl.ds(g * _G, _G)], rows_v.at[b], sem)
        pltpu.async_copy(lab_hbm.at[g], lab_v.at[b], sem)

    def _wait_load(g, b, sem):
        pltpu.make_async_copy(emb_hbm.at[pl.ds(g * _G, _G)],
                              rows_v.at[b], sem).wait()
        pltpu.make_async_copy(lab_hbm.at[g], lab_v.at[b], sem).wait()

    pl.when(g0 < g1)(lambda: _start_load(g0, 0, sem0))
    pl.when(g0 + 1 < g1)(lambda: _start_load(g0 + 1, 1, sem1))

    def _pair(p, _):
        for b in range(2):
            g = g0 + 2 * p + b

            def _do(g=g, b=b):
                _wait_load(g, b, sems[b])
                pltpu.sync_copy(rows_v.at[b], sums_sp.at[lab_v.at[b]], add=True)
                pltpu.sync_copy(ones_v, counts_sp.at[lab_v.at[b]], add=True)
                pl.when(g + 2 < g1)(lambda: _start_load(g + 2, b, sems[b]))

            pl.when(g < g1)(_do)
        return 0
    lax.fori_loop(0, (g1 - g0 + 1) // 2, _pair, 0)

    plsc.subcore_barrier()

    pltpu.sync_copy(sums_sp.at[pl.ds(base, _CPS)], stage_v)
    pltpu.sync_copy(stage_v, sums_out.at[c, pl.ds(base, _CPS)])
    pltpu.sync_copy(counts_sp.at[pl.ds(base, _CPS)], stage_v)
    pltpu.sync_copy(stage_v, counts_out.at[c, pl.ds(base, _CPS)])


def _phase1(embeddings, lab2d):
    ones = jnp.ones((_G, _DIM), jnp.float32)
    zero = jnp.zeros((_CPS, _W), jnp.float32)
    return _phase1_call(embeddings, lab2d, ones, zero)


_phase1_call = pl.kernel(
    _phase1_body,
    out_type=(
        jax.ShapeDtypeStruct((_NC, _CPAD, _W), jnp.float32),
        jax.ShapeDtypeStruct((_NC, _CPAD, _W), jnp.float32),
    ),
    mesh=plsc.VectorSubcoreMesh(
        core_axis_name="c", subcore_axis_name="s",
        num_cores=_NC, num_subcores=_NS),
    scratch_types=(
        pltpu.VMEM((2, _G, _W), jnp.float32),
        pltpu.VMEM((2, _G), jnp.int32),
        pltpu.VMEM((_G, _W), jnp.float32),
        pltpu.VMEM((_CPS, _W), jnp.float32),
        pltpu.VMEM_SHARED((_CPAD, _W), jnp.float32),
        pltpu.VMEM_SHARED((_CPAD, _W), jnp.float32),
        pltpu.SemaphoreType.DMA,
        pltpu.SemaphoreType.DMA,
    ),
)


def _combine_body(sums_ref, counts_ref, proto_ref, out_ref):
    total = sums_ref[0] + sums_ref[1]
    cnt = counts_ref[0, :, 0:1] + counts_ref[1, :, 0:1]
    mean = total / jnp.maximum(cnt, 1.0)
    proto = proto_ref[...]
    out_ref[...] = jnp.where(
        cnt > 0.0, _MOMENTUM * proto + (1.0 - _MOMENTUM) * mean, proto)


def kernel(embeddings, labels, prototypes):
    lab2d = labels.astype(jnp.int32).reshape(_NG, _G)
    sums, counts = _phase1(embeddings, lab2d)
    return pl.pallas_call(
        _combine_body,
        out_shape=jax.ShapeDtypeStruct((_NUM_CLASSES, _DIM), jnp.float32),
    )(sums[:, :_NUM_CLASSES], counts[:, :_NUM_CLASSES], prototypes)


# speedup vs baseline: 8.7767x; 1.1769x over previous
import jax
import jax.numpy as jnp
from jax import lax
from jax.experimental import pallas as pl
from jax.experimental.pallas import tpu as pltpu
from jax.experimental.pallas import tpu_sc as plsc

_NUM_CLASSES = 1000
_DIM = 128
_N = 320000
_MOMENTUM = 0.99

_NC = 2
_NS = 16
_NW = _NC * _NS
_G = 128
_NG = _N // _G
_CPAD = 1024
_CPS = _CPAD // _NS


def _phase1_body(emb_hbm, lab_hbm, labp_hbm, zero_hbm, zero1k_hbm,
                 sums_out, a_out, b_out,
                 rows_v, lab_v, labp_v, stage_v, a_v, b_v, sums_sp,
                 sem0, sem1):
    c = lax.axis_index("c")
    s = lax.axis_index("s")
    wid = s * _NC + c

    base = s * _CPS
    pltpu.sync_copy(zero_hbm, stage_v)
    pltpu.sync_copy(stage_v, sums_sp.at[pl.ds(base, _CPS)])
    pltpu.sync_copy(zero1k_hbm, a_v)
    pltpu.sync_copy(zero1k_hbm, b_v)
    plsc.subcore_barrier()

    g0 = wid * _NG // _NW
    g1 = (wid + 1) * _NG // _NW
    sems = (sem0, sem1)

    def _start_load(g, b, sem):
        pltpu.async_copy(emb_hbm.at[pl.ds(g * _G, _G)], rows_v.at[b], sem)
        pltpu.async_copy(lab_hbm.at[g], lab_v.at[b], sem)
        pltpu.async_copy(labp_hbm.at[g], labp_v.at[b], sem)

    def _wait_load(g, b, sem):
        pltpu.make_async_copy(emb_hbm.at[pl.ds(g * _G, _G)],
                              rows_v.at[b], sem).wait()
        pltpu.make_async_copy(lab_hbm.at[g], lab_v.at[b], sem).wait()
        pltpu.make_async_copy(labp_hbm.at[g], labp_v.at[b], sem).wait()

    pl.when(g0 < g1)(lambda: _start_load(g0, 0, sem0))
    pl.when(g0 + 1 < g1)(lambda: _start_load(g0 + 1, 1, sem1))

    iota16 = lax.iota(jnp.int32, 16)

    def _pair(p, _):
        for b in range(2):
            g = g0 + 2 * p + b

            def _do(g=g, b=b):
                _wait_load(g, b, sems[b])
                pltpu.sync_copy(rows_v.at[b], sums_sp.at[lab_v.at[b]], add=True)
                for k in range(8):
                    l = lab_v[b, pl.ds(16 * k, 16)]
                    lp = labp_v[b, pl.ds(16 * k, 16)]
                    bmask = l != lp
                    amask = jnp.logical_and(bmask, lp >= 0)
                    posf = (iota16 + (g * _G + 16 * k + 1)).astype(jnp.float32)
                    plsc.addupdate_scatter(b_v, [l], posf, mask=bmask)
                    plsc.addupdate_scatter(a_v, [lp], posf, mask=amask)
                pl.when(g + 2 < g1)(lambda: _start_load(g + 2, b, sems[b]))

            pl.when(g < g1)(_do)
        return 0
    lax.fori_loop(0, (g1 - g0 + 1) // 2, _pair, 0)

    plsc.subcore_barrier()

    pltpu.sync_copy(sums_sp.at[pl.ds(base, _CPS)], stage_v)
    pltpu.sync_copy(stage_v, sums_out.at[c, pl.ds(base, _CPS)])
    pltpu.sync_copy(a_v, a_out.at[wid])
    pltpu.sync_copy(b_v, b_out.at[wid])


def _phase1(embeddings, lab2d, labp2d):
    zero = jnp.zeros((_CPS, _DIM), jnp.float32)
    zero1k = jnp.zeros((_CPAD,), jnp.float32)
    return _phase1_call(embeddings, lab2d, labp2d, zero, zero1k)


_phase1_call = pl.kernel(
    _phase1_body,
    out_type=(
        jax.ShapeDtypeStruct((_NC, _CPAD, _DIM), jnp.float32),
        jax.ShapeDtypeStruct((_NW, _CPAD), jnp.float32),
        jax.ShapeDtypeStruct((_NW, _CPAD), jnp.float32),
    ),
    mesh=plsc.VectorSubcoreMesh(
        core_axis_name="c", subcore_axis_name="s",
        num_cores=_NC, num_subcores=_NS),
    compiler_params=pltpu.CompilerParams(needs_layout_passes=False),
    scratch_types=(
        pltpu.VMEM((2, _G, _DIM), jnp.float32),
        pltpu.VMEM((2, _G), jnp.int32),
        pltpu.VMEM((2, _G), jnp.int32),
        pltpu.VMEM((_CPS, _DIM), jnp.float32),
        pltpu.VMEM((_CPAD,), jnp.float32),
        pltpu.VMEM((_CPAD,), jnp.float32),
        pltpu.VMEM_SHARED((_CPAD, _DIM), jnp.float32),
        pltpu.SemaphoreType.DMA,
        pltpu.SemaphoreType.DMA,
    ),
)


def _combine_body(sums_ref, a_ref, b_ref, proto_ref, out_ref):
    total = sums_ref[0] + sums_ref[1]
    a = jnp.sum(a_ref[...], axis=0)
    bb = jnp.sum(b_ref[...], axis=0)
    cnt = jnp.where(a > 0.0, a - bb, (_N + 1.0) - bb)
    cnt = jnp.where(bb > 0.0, cnt, 0.0)
    mean = total / jnp.maximum(cnt, 1.0)
    proto = proto_ref[...]
    out_ref[...] = jnp.where(
        cnt > 0.0, _MOMENTUM * proto + (1.0 - _MOMENTUM) * mean, proto)


def kernel(embeddings, labels, prototypes):
    lab = labels.astype(jnp.int32)
    lab2d = lab.reshape(_NG, _G)
    labp2d = jnp.concatenate(
        [jnp.full((1,), -1, jnp.int32), lab[:-1]]).reshape(_NG, _G)
    sums, a, b = _phase1(embeddings, lab2d, labp2d)
    a3 = a[:, :_NUM_CLASSES].reshape(_NW, _NUM_CLASSES, 1)
    b3 = b[:, :_NUM_CLASSES].reshape(_NW, _NUM_CLASSES, 1)
    return pl.pallas_call(
        _combine_body,
        out_shape=jax.ShapeDtypeStruct((_NUM_CLASSES, _DIM), jnp.float32),
    )(sums[:, :_NUM_CLASSES], a3, b3, prototypes)


# speedup vs baseline: 8.9012x; 1.0142x over previous
import jax
import jax.numpy as jnp
from jax import lax
from jax.experimental import pallas as pl
from jax.experimental.pallas import tpu as pltpu
from jax.experimental.pallas import tpu_sc as plsc

_NUM_CLASSES = 1000
_DIM = 128
_N = 320000
_MOMENTUM = 0.99

_NC = 2
_NS = 16
_NW = _NC * _NS
_G = 128
_NG = _N // _G
_CPAD = 1024
_CPS = _CPAD // _NS


def _phase1_body(emb_hbm, lab_hbm, labp_hbm, zero_hbm, zero1k_hbm,
                 sums_out, a_out, b_out,
                 rows_v, lab_v, labp_v, stage_v, a_v, b_v, sums_sp,
                 sem0, sem1, ssem0, ssem1):
    c = lax.axis_index("c")
    s = lax.axis_index("s")
    wid = s * _NC + c

    base = s * _CPS
    pltpu.sync_copy(zero_hbm, stage_v)
    pltpu.sync_copy(stage_v, sums_sp.at[pl.ds(base, _CPS)])
    pltpu.sync_copy(zero1k_hbm, a_v)
    pltpu.sync_copy(zero1k_hbm, b_v)
    plsc.subcore_barrier()

    g0 = wid * _NG // _NW
    g1 = (wid + 1) * _NG // _NW
    sems = (sem0, sem1)
    ssems = (ssem0, ssem1)

    def _start_load(g, b, sem):
        pltpu.async_copy(emb_hbm.at[pl.ds(g * _G, _G)], rows_v.at[b], sem)
        pltpu.async_copy(lab_hbm.at[g], lab_v.at[b], sem)
        pltpu.async_copy(labp_hbm.at[g], labp_v.at[b], sem)

    def _wait_load(g, b, sem):
        pltpu.make_async_copy(emb_hbm.at[pl.ds(g * _G, _G)],
                              rows_v.at[b], sem).wait()
        pltpu.make_async_copy(lab_hbm.at[g], lab_v.at[b], sem).wait()
        pltpu.make_async_copy(labp_hbm.at[g], labp_v.at[b], sem).wait()

    pl.when(g0 < g1)(lambda: _start_load(g0, 0, sem0))
    pl.when(g0 + 1 < g1)(lambda: _start_load(g0 + 1, 1, sem1))

    iota16 = lax.iota(jnp.int32, 16)

    def _pair(p, _):
        for b in range(2):
            g = g0 + 2 * p + b

            def _do(g=g, b=b):
                _wait_load(g, b, sems[b])
                cp = pltpu.async_copy(
                    rows_v.at[b], sums_sp.at[lab_v.at[b]], ssems[b], add=True)
                for k in range(8):
                    l = lab_v[b, pl.ds(16 * k, 16)]
                    lp = labp_v[b, pl.ds(16 * k, 16)]
                    bmask = l != lp
                    amask = jnp.logical_and(bmask, lp >= 0)
                    posf = (iota16 + (g * _G + 16 * k + 1)).astype(jnp.float32)
                    plsc.addupdate_scatter(b_v, [l], posf, mask=bmask)
                    plsc.addupdate_scatter(a_v, [lp], posf, mask=amask)
                cp.wait()
                pl.when(g + 2 < g1)(lambda: _start_load(g + 2, b, sems[b]))

            pl.when(g < g1)(_do)
        return 0
    lax.fori_loop(0, (g1 - g0 + 1) // 2, _pair, 0)

    plsc.subcore_barrier()

    pltpu.sync_copy(sums_sp.at[pl.ds(base, _CPS)], stage_v)
    pltpu.sync_copy(stage_v, sums_out.at[c, pl.ds(base, _CPS)])
    pltpu.sync_copy(a_v, a_out.at[wid])
    pltpu.sync_copy(b_v, b_out.at[wid])


def _phase1(embeddings, lab2d, labp2d):
    zero = jnp.zeros((_CPS, _DIM), jnp.float32)
    zero1k = jnp.zeros((_CPAD,), jnp.float32)
    return _phase1_call(embeddings, lab2d, labp2d, zero, zero1k)


_phase1_call = pl.kernel(
    _phase1_body,
    out_type=(
        jax.ShapeDtypeStruct((_NC, _CPAD, _DIM), jnp.float32),
        jax.ShapeDtypeStruct((_NW, _CPAD), jnp.float32),
        jax.ShapeDtypeStruct((_NW, _CPAD), jnp.float32),
    ),
    mesh=plsc.VectorSubcoreMesh(
        core_axis_name="c", subcore_axis_name="s",
        num_cores=_NC, num_subcores=_NS),
    compiler_params=pltpu.CompilerParams(needs_layout_passes=False),
    scratch_types=(
        pltpu.VMEM((2, _G, _DIM), jnp.float32),
        pltpu.VMEM((2, _G), jnp.int32),
        pltpu.VMEM((2, _G), jnp.int32),
        pltpu.VMEM((_CPS, _DIM), jnp.float32),
        pltpu.VMEM((_CPAD,), jnp.float32),
        pltpu.VMEM((_CPAD,), jnp.float32),
        pltpu.VMEM_SHARED((_CPAD, _DIM), jnp.float32),
        pltpu.SemaphoreType.DMA,
        pltpu.SemaphoreType.DMA,
        pltpu.SemaphoreType.DMA,
        pltpu.SemaphoreType.DMA,
    ),
)


def _combine_body(sums_ref, a_ref, b_ref, proto_ref, out_ref):
    total = sums_ref[0] + sums_ref[1]
    a = jnp.sum(a_ref[...], axis=0)
    bb = jnp.sum(b_ref[...], axis=0)
    cnt = jnp.where(a > 0.0, a - bb, (_N + 1.0) - bb)
    cnt = jnp.where(bb > 0.0, cnt, 0.0)
    mean = total / jnp.maximum(cnt, 1.0)
    proto = proto_ref[...]
    out_ref[...] = jnp.where(
        cnt > 0.0, _MOMENTUM * proto + (1.0 - _MOMENTUM) * mean, proto)


def kernel(embeddings, labels, prototypes):
    lab = labels.astype(jnp.int32)
    lab2d = lab.reshape(_NG, _G)
    labp2d = jnp.concatenate(
        [jnp.full((1,), -1, jnp.int32), lab[:-1]]).reshape(_NG, _G)
    sums, a, b = _phase1(embeddings, lab2d, labp2d)
    a3 = a[:, :_NUM_CLASSES].reshape(_NW, _NUM_CLASSES, 1)
    b3 = b[:, :_NUM_CLASSES].reshape(_NW, _NUM_CLASSES, 1)
    return pl.pallas_call(
        _combine_body,
        out_shape=jax.ShapeDtypeStruct((_NUM_CLASSES, _DIM), jnp.float32),
    )(sums[:, :_NUM_CLASSES], a3, b3, prototypes)
